# baseline (speedup 1.0000x reference)
import jax
import jax.numpy as jnp
from jax import lax
from jax.experimental import pallas as pl
from jax.experimental.pallas import tpu as pltpu
from jax.experimental.pallas import tpu_sc as plsc

N = 10000
E = 320000
H = 128
NG = 64
L = 6

NC = 2
NS = 16
NW = NC * NS
EPT = E // NW
K = 80
NCHUNK = EPT // K
NP = 10240
ROWS_PER_TILE = NP // NS

LOG2 = 0.6931471805599453
HI = lax.Precision.HIGHEST


def _wmlp_body(z1_ref, ivw_ref, ivb_ref, w1_ref, b1_ref, w2_ref, b2_ref,
               out_ref):
    u = jnp.dot(ivw_ref[...], w1_ref[...],
                preferred_element_type=jnp.float32, precision=HI)
    c = jnp.dot(ivb_ref[...], w1_ref[...],
                preferred_element_type=jnp.float32, precision=HI) + b1_ref[...]
    h = jnp.maximum(z1_ref[...] * u + c, 0.0)
    out_ref[...] = jnp.dot(h, w2_ref[...],
                           preferred_element_type=jnp.float32,
                           precision=HI) + b2_ref[...]


def _edge_weights(z1c, ivw, ivb, w1l, b1l, w2l, b2l):
    B = 2000
    return pl.pallas_call(
        _wmlp_body,
        grid=(E // B,),
        in_specs=[
            pl.BlockSpec((B, 1), lambda e: (e, 0)),
            pl.BlockSpec((1, 50), lambda e: (0, 0)),
            pl.BlockSpec((1, 50), lambda e: (0, 0)),
            pl.BlockSpec((50, H), lambda e: (0, 0)),
            pl.BlockSpec((1, H), lambda e: (0, 0)),
            pl.BlockSpec((H, H), lambda e: (0, 0)),
            pl.BlockSpec((1, H), lambda e: (0, 0)),
        ],
        out_specs=pl.BlockSpec((B, H), lambda e: (e, 0)),
        out_shape=jax.ShapeDtypeStruct((E, H), jnp.float32),
    )(z1c, ivw, ivb, w1l, b1l, w2l, b2l)


def _init_body(z0_ref, embp_ref, lin0_ref, out_ref):
    io = lax.broadcasted_iota(jnp.int32, (1, H), 1)
    onehot = (z0_ref[...] == io).astype(jnp.float32)
    v0 = jnp.dot(onehot, embp_ref[...],
                 preferred_element_type=jnp.float32, precision=HI)
    out_ref[...] = jnp.dot(v0, lin0_ref[...],
                           preferred_element_type=jnp.float32, precision=HI)


def _init_x(z0c, embp, lin0):
    B = 2000
    return pl.pallas_call(
        _init_body,
        grid=(N // B,),
        in_specs=[
            pl.BlockSpec((B, 1), lambda i: (i, 0)),
            pl.BlockSpec((H, H), lambda i: (0, 0)),
            pl.BlockSpec((H, H), lambda i: (0, 0)),
        ],
        out_specs=pl.BlockSpec((B, H), lambda i: (i, 0)),
        out_shape=jax.ShapeDtypeStruct((N, H), jnp.float32),
    )(z0c, embp, lin0)


def _combine_body(p_ref, bias_ref, lin_ref, out_ref):
    v = jnp.maximum(p_ref[0] + p_ref[1] + bias_ref[...], 0.0)
    out_ref[...] = jnp.dot(v, lin_ref[...],
                           preferred_element_type=jnp.float32, precision=HI)


def _combine(p, biasl, linn):
    B = 2000
    return pl.pallas_call(
        _combine_body,
        grid=(N // B,),
        in_specs=[
            pl.BlockSpec((2, B, H), lambda i: (0, i, 0)),
            pl.BlockSpec((1, H), lambda i: (0, 0)),
            pl.BlockSpec((H, H), lambda i: (0, 0)),
        ],
        out_specs=pl.BlockSpec((B, H), lambda i: (i, 0)),
        out_shape=jax.ShapeDtypeStruct((N, H), jnp.float32),
    )(p, biasl, linn)


def _head_body(p_ref, bias_ref, cw1_ref, cb1_ref, cw2_ref, cb2_ref,
               batch_ref, out_ref):
    v0 = jnp.maximum(p_ref[0] + p_ref[1] + bias_ref[...], 0.0)
    t = jnp.dot(v0, cw1_ref[...],
                preferred_element_type=jnp.float32, precision=HI) + cb1_ref[...]
    sp = jnp.maximum(t, 0.0) + jnp.log(1.0 + jnp.exp(-jnp.abs(t))) - LOG2
    y = jnp.dot(sp, cw2_ref[...],
                preferred_element_type=jnp.float32, precision=HI) + cb2_ref[...]
    io = lax.broadcasted_iota(jnp.int32, (1, H), 1)
    onehot = (batch_ref[...] == io).astype(jnp.float32)
    out_ref[...] = jnp.sum(onehot * y, axis=0, keepdims=True)


def _head(p, biasl, cw1p, cb1p, cw2p, cb2p, batc):
    return pl.pallas_call(
        _head_body,
        grid=(1,),
        in_specs=[
            pl.BlockSpec((2, N, H), lambda i: (0, 0, 0)),
            pl.BlockSpec((1, H), lambda i: (0, 0)),
            pl.BlockSpec((H, H), lambda i: (0, 0)),
            pl.BlockSpec((1, H), lambda i: (0, 0)),
            pl.BlockSpec((H, 1), lambda i: (0, 0)),
            pl.BlockSpec((1, 1), lambda i: (0, 0)),
            pl.BlockSpec((N, 1), lambda i: (0, 0)),
        ],
        out_specs=pl.BlockSpec((1, H), lambda i: (0, 0)),
        out_shape=jax.ShapeDtypeStruct((1, H), jnp.float32),
    )(p, biasl, cw1p, cb1p, cw2p, cb2p, batc)


def _edge_body(x_hbm, w_hbm, i0_hbm, i1_hbm, z_hbm, out_hbm,
               acc, idx0_v, idx1_v, xa, xb, wv, sem0, sem1, sem2):
    cid = lax.axis_index("c")
    sid = lax.axis_index("s")
    wid = cid * NS + sid
    row0 = sid * ROWS_PER_TILE
    pltpu.sync_copy(z_hbm.at[pl.ds(row0, ROWS_PER_TILE)],
                    acc.at[pl.ds(row0, ROWS_PER_TILE)])
    plsc.subcore_barrier()

    base0 = wid * EPT

    @pl.loop(0, NCHUNK)
    def _chunk(g):
        base = base0 + g * K
        pltpu.sync_copy(i0_hbm.at[pl.ds(base, K)], idx0_v)
        pltpu.sync_copy(i1_hbm.at[pl.ds(base, K)], idx1_v)
        cpa = pltpu.async_copy(x_hbm.at[idx1_v], xa, sem0)
        cpb = pltpu.async_copy(x_hbm.at[idx0_v], xb, sem1)
        cpw = pltpu.async_copy(w_hbm.at[pl.ds(base, K)], wv, sem2)
        cpa.wait()
        cpb.wait()
        cpw.wait()

        @pl.loop(0, K)
        def _mul(r):
            for c in range(H // 16):
                rs = pl.ds(r, 1)
                cs = pl.ds(c * 16, 16)
                wvec = wv[rs, cs]
                xa[rs, cs] = xa[rs, cs] * wvec
                xb[rs, cs] = xb[rs, cs] * wvec

        pltpu.sync_copy(xa, acc.at[idx0_v], add=True)
        pltpu.sync_copy(xb, acc.at[idx1_v], add=True)

    plsc.subcore_barrier()
    pltpu.sync_copy(acc.at[pl.ds(row0, ROWS_PER_TILE)],
                    out_hbm.at[cid, pl.ds(row0, ROWS_PER_TILE)])


def _edge_pass(x, w, i0, i1, zeros_nh):
    mesh = plsc.VectorSubcoreMesh(core_axis_name="c", subcore_axis_name="s")
    f = pl.kernel(
        _edge_body,
        out_type=jax.ShapeDtypeStruct((NC, NP, H), jnp.float32),
        mesh=mesh,
        scratch_types=[
            pltpu.VMEM_SHARED((NP, H), jnp.float32),
            pltpu.VMEM((K,), jnp.int32),
            pltpu.VMEM((K,), jnp.int32),
            pltpu.VMEM((K, H), jnp.float32),
            pltpu.VMEM((K, H), jnp.float32),
            pltpu.VMEM((K, H), jnp.float32),
            pltpu.SemaphoreType.DMA,
            pltpu.SemaphoreType.DMA,
            pltpu.SemaphoreType.DMA,
        ],
    )
    return f(x, w, i0, i1, zeros_nh)


def kernel(z0, z1, z2, z3, batch, edge_index0, edge_index1, edge_index2,
           emb_table, iv_w1, iv_b1, iv_w2, iv_b2, iv_w3, iv_b3,
           lin_ws, biases, mlp_w1, mlp_b1, mlp_w2, mlp_b2,
           c_w1, c_b1, c_w2, c_b2):
    f32 = jnp.float32
    z1c = z1.reshape(E, 1).astype(f32)
    ivw = iv_w1.astype(f32)
    ivb = iv_b1.reshape(1, 50).astype(f32)
    embp = jnp.zeros((H, H), f32).at[:100, :].set(emb_table)
    zeros_nh = jnp.zeros((NP, H), f32)
    ei = edge_index0.astype(jnp.int32)
    i0 = ei[0]
    i1 = ei[1]
    z0c = z0.astype(jnp.int32).reshape(N, 1)
    batc = batch.astype(jnp.int32).reshape(N, 1)
    cw1p = jnp.zeros((H, H), f32).at[:, :NG].set(c_w1)
    cb1p = jnp.zeros((1, H), f32).at[0, :NG].set(c_b1)
    cw2p = jnp.zeros((H, 1), f32).at[:NG, :].set(c_w2)
    cb2p = c_b2.reshape(1, 1).astype(f32)

    ws = [
        _edge_weights(z1c, ivw, ivb, mlp_w1[l], mlp_b1[l].reshape(1, H),
                      mlp_w2[l], mlp_b2[l].reshape(1, H))
        for l in range(L)
    ]
    x = _init_x(z0c, embp, lin_ws[0])
    for l in range(L):
        p = _edge_pass(x, ws[l], i0, i1, zeros_nh)
        if l < L - 1:
            x = _combine(p, biases[l].reshape(1, H), lin_ws[l + 1])
        else:
            r = _head(p, biases[l].reshape(1, H), cw1p, cb1p, cw2p, cb2p,
                      batc)
    return r[0, :NG].reshape(NG, 1)


# speedup vs baseline: 1.1735x; 1.1735x over previous
import jax
import jax.numpy as jnp
from jax import lax
from jax.experimental import pallas as pl
from jax.experimental.pallas import tpu as pltpu
from jax.experimental.pallas import tpu_sc as plsc

N = 10000
E = 320000
H = 128
NG = 64
L = 6

NC = 2
NS = 16
NW = NC * NS
EPT = E // NW
K = 80
NCHUNK = EPT // K
NP = 10240
ROWS_PER_TILE = NP // NS

LOG2 = 0.6931471805599453
HI = lax.Precision.HIGHEST


def _wmlp_body(z1_ref, ivw_ref, ivb_ref, w1_ref, b1_ref, w2_ref, b2_ref,
               out_ref):
    u = jnp.dot(ivw_ref[...], w1_ref[...],
                preferred_element_type=jnp.float32, precision=HI)
    c = jnp.dot(ivb_ref[...], w1_ref[...],
                preferred_element_type=jnp.float32, precision=HI) + b1_ref[...]
    h = jnp.maximum(z1_ref[...] * u + c, 0.0)
    out_ref[...] = jnp.dot(h, w2_ref[...],
                           preferred_element_type=jnp.float32,
                           precision=HI) + b2_ref[...]


def _edge_weights(z1c, ivw, ivb, w1l, b1l, w2l, b2l):
    B = 2000
    return pl.pallas_call(
        _wmlp_body,
        grid=(E // B,),
        in_specs=[
            pl.BlockSpec((B, 1), lambda e: (e, 0)),
            pl.BlockSpec((1, 50), lambda e: (0, 0)),
            pl.BlockSpec((1, 50), lambda e: (0, 0)),
            pl.BlockSpec((50, H), lambda e: (0, 0)),
            pl.BlockSpec((1, H), lambda e: (0, 0)),
            pl.BlockSpec((H, H), lambda e: (0, 0)),
            pl.BlockSpec((1, H), lambda e: (0, 0)),
        ],
        out_specs=pl.BlockSpec((B, H), lambda e: (e, 0)),
        out_shape=jax.ShapeDtypeStruct((E, H), jnp.float32),
    )(z1c, ivw, ivb, w1l, b1l, w2l, b2l)


def _init_body(z0_ref, embp_ref, lin0_ref, out_ref):
    io = lax.broadcasted_iota(jnp.int32, (1, H), 1)
    onehot = (z0_ref[...] == io).astype(jnp.float32)
    v0 = jnp.dot(onehot, embp_ref[...],
                 preferred_element_type=jnp.float32, precision=HI)
    out_ref[...] = jnp.dot(v0, lin0_ref[...],
                           preferred_element_type=jnp.float32, precision=HI)


def _init_x(z0c, embp, lin0):
    B = 2000
    return pl.pallas_call(
        _init_body,
        grid=(N // B,),
        in_specs=[
            pl.BlockSpec((B, 1), lambda i: (i, 0)),
            pl.BlockSpec((H, H), lambda i: (0, 0)),
            pl.BlockSpec((H, H), lambda i: (0, 0)),
        ],
        out_specs=pl.BlockSpec((B, H), lambda i: (i, 0)),
        out_shape=jax.ShapeDtypeStruct((N, H), jnp.float32),
    )(z0c, embp, lin0)


def _combine_body(p_ref, bias_ref, lin_ref, out_ref):
    v = jnp.maximum(p_ref[0] + p_ref[1] + bias_ref[...], 0.0)
    out_ref[...] = jnp.dot(v, lin_ref[...],
                           preferred_element_type=jnp.float32, precision=HI)


def _combine(p, biasl, linn):
    B = 2000
    return pl.pallas_call(
        _combine_body,
        grid=(N // B,),
        in_specs=[
            pl.BlockSpec((2, B, H), lambda i: (0, i, 0)),
            pl.BlockSpec((1, H), lambda i: (0, 0)),
            pl.BlockSpec((H, H), lambda i: (0, 0)),
        ],
        out_specs=pl.BlockSpec((B, H), lambda i: (i, 0)),
        out_shape=jax.ShapeDtypeStruct((N, H), jnp.float32),
    )(p, biasl, linn)


def _head_body(p_ref, bias_ref, cw1_ref, cb1_ref, cw2_ref, cb2_ref,
               batch_ref, out_ref):
    v0 = jnp.maximum(p_ref[0] + p_ref[1] + bias_ref[...], 0.0)
    t = jnp.dot(v0, cw1_ref[...],
                preferred_element_type=jnp.float32, precision=HI) + cb1_ref[...]
    sp = jnp.maximum(t, 0.0) + jnp.log(1.0 + jnp.exp(-jnp.abs(t))) - LOG2
    y = jnp.dot(sp, cw2_ref[...],
                preferred_element_type=jnp.float32, precision=HI) + cb2_ref[...]
    io = lax.broadcasted_iota(jnp.int32, (1, H), 1)
    onehot = (batch_ref[...] == io).astype(jnp.float32)
    out_ref[...] = jnp.sum(onehot * y, axis=0, keepdims=True)


def _head(p, biasl, cw1p, cb1p, cw2p, cb2p, batc):
    return pl.pallas_call(
        _head_body,
        grid=(1,),
        in_specs=[
            pl.BlockSpec((2, N, H), lambda i: (0, 0, 0)),
            pl.BlockSpec((1, H), lambda i: (0, 0)),
            pl.BlockSpec((H, H), lambda i: (0, 0)),
            pl.BlockSpec((1, H), lambda i: (0, 0)),
            pl.BlockSpec((H, 1), lambda i: (0, 0)),
            pl.BlockSpec((1, 1), lambda i: (0, 0)),
            pl.BlockSpec((N, 1), lambda i: (0, 0)),
        ],
        out_specs=pl.BlockSpec((1, H), lambda i: (0, 0)),
        out_shape=jax.ShapeDtypeStruct((1, H), jnp.float32),
    )(p, biasl, cw1p, cb1p, cw2p, cb2p, batc)


def _edge_body(x_hbm, w_hbm, icat_hbm, z_hbm, out_hbm,
               acc, idxc, xa, xb, wv, sem0, sem1, sem2, sem3):
    cid = lax.axis_index("c")
    sid = lax.axis_index("s")
    wid = cid * NS + sid
    row0 = sid * ROWS_PER_TILE
    pltpu.sync_copy(z_hbm.at[pl.ds(row0, ROWS_PER_TILE)],
                    acc.at[pl.ds(row0, ROWS_PER_TILE)])
    plsc.subcore_barrier()

    base0 = wid * EPT

    def wait_out():
        pltpu.make_async_copy(xa, acc.at[idxc.at[0, 0]], sem3).wait()
        pltpu.make_async_copy(xb, acc.at[idxc.at[0, 0]], sem3).wait()

    @pl.loop(0, NCHUNK)
    def _chunk(g):
        base = base0 + g * K
        b = g % 2
        pltpu.sync_copy(icat_hbm.at[wid * NCHUNK + g], idxc.at[b])

        @pl.when(g > 0)
        def _():
            wait_out()

        cpa = pltpu.async_copy(x_hbm.at[idxc.at[b, 1]], xa, sem0)
        cpb = pltpu.async_copy(x_hbm.at[idxc.at[b, 0]], xb, sem1)
        cpw = pltpu.async_copy(w_hbm.at[pl.ds(base, K)], wv, sem2)
        cpa.wait()
        cpb.wait()
        cpw.wait()

        @plsc.parallel_loop(0, K, unroll=2)
        def _mul(r):
            rs = pl.ds(r, 1)
            for c in range(H // 16):
                cs = pl.ds(c * 16, 16)
                wvec = wv[rs, cs]
                xa[rs, cs] = xa[rs, cs] * wvec
                xb[rs, cs] = xb[rs, cs] * wvec

        pltpu.async_copy(xa, acc.at[idxc.at[b, 0]], sem3, add=True)
        pltpu.async_copy(xb, acc.at[idxc.at[b, 1]], sem3, add=True)

    wait_out()
    plsc.subcore_barrier()
    pltpu.sync_copy(acc.at[pl.ds(row0, ROWS_PER_TILE)],
                    out_hbm.at[cid, pl.ds(row0, ROWS_PER_TILE)])


def _edge_pass(x, w, icat, zeros_nh):
    mesh = plsc.VectorSubcoreMesh(core_axis_name="c", subcore_axis_name="s")
    f = pl.kernel(
        _edge_body,
        out_type=jax.ShapeDtypeStruct((NC, NP, H), jnp.float32),
        mesh=mesh,
        scratch_types=[
            pltpu.VMEM_SHARED((NP, H), jnp.float32),
            pltpu.VMEM((2, 2, K), jnp.int32),
            pltpu.VMEM((K, H), jnp.float32),
            pltpu.VMEM((K, H), jnp.float32),
            pltpu.VMEM((K, H), jnp.float32),
            pltpu.SemaphoreType.DMA,
            pltpu.SemaphoreType.DMA,
            pltpu.SemaphoreType.DMA,
            pltpu.SemaphoreType.DMA,
        ],
    )
    return f(x, w, icat, zeros_nh)


def kernel(z0, z1, z2, z3, batch, edge_index0, edge_index1, edge_index2,
           emb_table, iv_w1, iv_b1, iv_w2, iv_b2, iv_w3, iv_b3,
           lin_ws, biases, mlp_w1, mlp_b1, mlp_w2, mlp_b2,
           c_w1, c_b1, c_w2, c_b2):
    f32 = jnp.float32
    z1c = z1.reshape(E, 1).astype(f32)
    ivw = iv_w1.astype(f32)
    ivb = iv_b1.reshape(1, 50).astype(f32)
    embp = jnp.zeros((H, H), f32).at[:100, :].set(emb_table)
    zeros_nh = jnp.zeros((NP, H), f32)
    ei = edge_index0.astype(jnp.int32)
    icat = jnp.stack([ei[0].reshape(-1, K), ei[1].reshape(-1, K)], axis=1)
    z0c = z0.astype(jnp.int32).reshape(N, 1)
    batc = batch.astype(jnp.int32).reshape(N, 1)
    cw1p = jnp.zeros((H, H), f32).at[:, :NG].set(c_w1)
    cb1p = jnp.zeros((1, H), f32).at[0, :NG].set(c_b1)
    cw2p = jnp.zeros((H, 1), f32).at[:NG, :].set(c_w2)
    cb2p = c_b2.reshape(1, 1).astype(f32)

    ws = [
        _edge_weights(z1c, ivw, ivb, mlp_w1[l], mlp_b1[l].reshape(1, H),
                      mlp_w2[l], mlp_b2[l].reshape(1, H))
        for l in range(L)
    ]
    x = _init_x(z0c, embp, lin_ws[0])
    for l in range(L):
        p = _edge_pass(x, ws[l], icat, zeros_nh)
        if l < L - 1:
            x = _combine(p, biases[l].reshape(1, H), lin_ws[l + 1])
        else:
            r = _head(p, biases[l].reshape(1, H), cw1p, cb1p, cw2p, cb2p,
                      batc)
    return r[0, :NG].reshape(NG, 1)
